# Initial kernel scaffold; baseline (speedup 1.0000x reference)
import jax
import jax.numpy as jnp
from jax.experimental import pallas as pl


def kernel(x, hyperedge_index, temb, W, b):
    raise NotImplementedError("write your pallas kernel here")


# speedup vs baseline: 13.4076x; 13.4076x over previous
import functools

import jax
import jax.numpy as jnp
from jax import lax
from jax.experimental import pallas as pl
from jax.experimental.pallas import tpu as pltpu
from jax.experimental.pallas import tpu_sc as plsc

N_NODES = 10000
N_EDGES = 10000
N_INC = 320000
D = 128
DP = 144

NW = 32
PER_W = N_INC // NW
BLK = 80
NBLK = PER_W // BLK
ACC_ROWS = 10240
STRIPE = ACC_ROWS // 16
CH = 128

_MESH = plsc.VectorSubcoreMesh(core_axis_name="c", subcore_axis_name="s")


def _mm_body(x_ref, w_ref, oh_ref, o_ref):
    o_ref[...] = (
        jnp.dot(x_ref[...], w_ref[...], preferred_element_type=jnp.float32)
        + oh_ref[...]
    )


def _matmul_ext(x, w_ext, oh):
    return pl.pallas_call(
        _mm_body,
        grid=(10,),
        in_specs=[
            pl.BlockSpec((1000, D), lambda i: (i, 0)),
            pl.BlockSpec((D, DP), lambda i: (0, 0)),
            pl.BlockSpec((1, DP), lambda i: (0, 0)),
        ],
        out_specs=pl.BlockSpec((1000, DP), lambda i: (i, 0)),
        out_shape=jax.ShapeDtypeStruct((N_NODES, DP), jnp.float32),
    )(x, w_ext, oh)


def _sc_body(table, gidx, sidx, zeros, out, gblk_v, sblk_v, rows_v, wb_v, acc, sem):
    c = lax.axis_index("c")
    s = lax.axis_index("s")
    wid = s * 2 + c

    pltpu.sync_copy(zeros, wb_v)
    for k in range(STRIPE // CH):
        pltpu.sync_copy(wb_v, acc.at[pl.ds(s * STRIPE + k * CH, CH)])
    plsc.subcore_barrier()

    def blk(j, carry):
        pltpu.sync_copy(gidx.at[wid, j], gblk_v)
        pltpu.sync_copy(sidx.at[wid, j], sblk_v)
        pltpu.async_copy(table.at[gblk_v], rows_v, sem).wait()
        pltpu.sync_copy(rows_v, acc.at[sblk_v], add=True)
        return carry

    lax.fori_loop(0, NBLK, blk, 0)
    plsc.subcore_barrier()

    for k in range(STRIPE // CH):
        pltpu.sync_copy(acc.at[pl.ds(s * STRIPE + k * CH, CH)], wb_v)
        pltpu.sync_copy(wb_v, out.at[c, s, pl.ds(k * CH, CH)])


def _sc_pass(table, gidx, sidx, zeros):
    return pl.kernel(
        _sc_body,
        out_type=jax.ShapeDtypeStruct((2, 16, STRIPE, DP), jnp.float32),
        mesh=_MESH,
        scratch_types=[
            pltpu.VMEM((BLK,), jnp.int32),
            pltpu.VMEM((BLK,), jnp.int32),
            pltpu.VMEM((BLK, DP), jnp.float32),
            pltpu.VMEM((CH, DP), jnp.float32),
            pltpu.VMEM_SHARED((ACC_ROWS, DP), jnp.float32),
            pltpu.SemaphoreType.DMA,
        ],
        compiler_params=pltpu.CompilerParams(use_tc_tiling_on_sc=False),
    )(table, gidx, sidx, zeros)


def _c1_body(p_ref, sel_ref, mask_ref, oh_ref, o_ref):
    t = p_ref[0] + p_ref[1]
    cnt = jnp.sum(t * sel_ref[...], axis=1, keepdims=True)
    inv = 1.0 / jnp.maximum(cnt, 1.0)
    o_ref[...] = t * inv * mask_ref[...] + oh_ref[...]


def _combine_mid(p, sel, mask, oh):
    return pl.pallas_call(
        _c1_body,
        grid=(10,),
        in_specs=[
            pl.BlockSpec((2, 1000, DP), lambda i: (0, i, 0)),
            pl.BlockSpec((1, DP), lambda i: (0, 0)),
            pl.BlockSpec((1, DP), lambda i: (0, 0)),
            pl.BlockSpec((1, DP), lambda i: (0, 0)),
        ],
        out_specs=pl.BlockSpec((1000, DP), lambda i: (i, 0)),
        out_shape=jax.ShapeDtypeStruct((N_NODES, DP), jnp.float32),
    )(p, sel, mask, oh)


def _c2_body(q_ref, sel_ref, b_ref, o_ref):
    t = q_ref[0] + q_ref[1]
    cnt = jnp.sum(t * sel_ref[...], axis=1, keepdims=True)
    inv = 1.0 / jnp.maximum(cnt, 1.0)
    t = t * inv
    o_ref[...] = t[:, :D] + b_ref[...]


def _combine_out(q, sel, b2):
    return pl.pallas_call(
        _c2_body,
        grid=(10,),
        in_specs=[
            pl.BlockSpec((2, 1000, DP), lambda i: (0, i, 0)),
            pl.BlockSpec((1, DP), lambda i: (0, 0)),
            pl.BlockSpec((1, D), lambda i: (0, 0)),
        ],
        out_specs=pl.BlockSpec((1000, D), lambda i: (i, 0)),
        out_shape=jax.ShapeDtypeStruct((N_NODES, D), jnp.float32),
    )(q, sel, b2)


def kernel(x, hyperedge_index, temb, W, b):
    del temb
    hidx = hyperedge_index.astype(jnp.int32)
    nidx = hidx[0].reshape(NW, NBLK, BLK)
    eidx = hidx[1].reshape(NW, NBLK, BLK)

    w_ext = jnp.pad(W, ((0, 0), (0, DP - D)))
    oh = jnp.zeros((1, DP), jnp.float32).at[0, D].set(1.0)
    mask = jnp.pad(jnp.ones((1, D), jnp.float32), ((0, 0), (0, DP - D)))
    zeros = jnp.zeros((CH, DP), jnp.float32)

    xw = _matmul_ext(x, w_ext, oh)
    p = _sc_pass(xw, nidx, eidx, zeros).reshape(2, ACC_ROWS, DP)
    ef = _combine_mid(p, oh, mask, oh)
    q = _sc_pass(ef, eidx, nidx, zeros).reshape(2, ACC_ROWS, DP)
    return _combine_out(q, oh, b.reshape(1, D))


# speedup vs baseline: 20.3712x; 1.5194x over previous
import jax
import jax.numpy as jnp
from jax import lax
from jax.experimental import pallas as pl
from jax.experimental.pallas import tpu as pltpu
from jax.experimental.pallas import tpu_sc as plsc

N_NODES = 10000
N_INC = 320000
D = 128
DP = 160
HW = 80
CCOL = 48

NT = 16
PER_T = N_INC // NT
BLK = 80
NBLK = PER_T // BLK
NBUF = 5
STRIPE = N_NODES // NT
CH = 25

_MESH = plsc.VectorSubcoreMesh(core_axis_name="c", subcore_axis_name="s")


def _mm_body(x_ref, w_ref, oh_ref, o_ref):
    o_ref[...] = (
        jnp.dot(x_ref[...], w_ref[...], preferred_element_type=jnp.float32)
        + oh_ref[...]
    )


def _matmul_ext(x, w_ext, oh):
    return pl.pallas_call(
        _mm_body,
        grid=(10,),
        in_specs=[
            pl.BlockSpec((1000, D), lambda i: (i, 0)),
            pl.BlockSpec((D, DP), lambda i: (0, 0)),
            pl.BlockSpec((1, DP), lambda i: (0, 0)),
        ],
        out_specs=pl.BlockSpec((1000, DP), lambda i: (i, 0)),
        out_shape=jax.ShapeDtypeStruct((N_NODES, DP), jnp.float32),
    )(x, w_ext, oh)


def _sc_body(table, gidx, sidx, zeros, out,
             gidx_v, sidx_v, r0, r1, r2, r3, r4, wb_v, acc,
             s0, s1, s2, s3, s4):
    rows = (r0, r1, r2, r3, r4)
    sems = (s0, s1, s2, s3, s4)
    c = lax.axis_index("c")
    s = lax.axis_index("s")
    half = table.at[c]

    pltpu.sync_copy(zeros, wb_v)
    for k in range(STRIPE // CH):
        pltpu.sync_copy(wb_v, acc.at[pl.ds(s * STRIPE + k * CH, CH)])

    pltpu.sync_copy(gidx.at[s], gidx_v)
    pltpu.sync_copy(sidx.at[s], sidx_v)
    plsc.subcore_barrier()

    def grp(g, carry):
        base = g * NBUF
        cps = [
            pltpu.async_copy(half.at[gidx_v.at[base + b]], rows[b], sems[b])
            for b in range(NBUF)
        ]
        for b in range(NBUF):
            cps[b].wait()
            pltpu.sync_copy(rows[b], acc.at[sidx_v.at[base + b]], add=True)
        return carry

    lax.fori_loop(0, NBLK // NBUF, grp, 0)
    plsc.subcore_barrier()

    for k in range(STRIPE // CH):
        pltpu.sync_copy(acc.at[pl.ds(s * STRIPE + k * CH, CH)], wb_v)
        pltpu.sync_copy(wb_v, out.at[c, s, pl.ds(k * CH, CH)])


def _sc_pass(table, gidx, sidx, zeros):
    return pl.kernel(
        _sc_body,
        out_type=jax.ShapeDtypeStruct((2, NT, STRIPE, HW), jnp.float32),
        mesh=_MESH,
        scratch_types=[
            pltpu.VMEM((NBLK, BLK), jnp.int32),
            pltpu.VMEM((NBLK, BLK), jnp.int32),
        ] + [pltpu.VMEM((BLK, HW), jnp.float32) for _ in range(NBUF)] + [
            pltpu.VMEM((CH, HW), jnp.float32),
            pltpu.VMEM_SHARED((N_NODES, HW), jnp.float32),
        ] + [pltpu.SemaphoreType.DMA for _ in range(NBUF)],
        compiler_params=pltpu.CompilerParams(use_tc_tiling_on_sc=False),
    )(table, gidx, sidx, zeros)


def _c1_body(p_ref, sel_ref, mask_ref, oh_ref, o_ref):
    t1 = p_ref[1]
    cnt = jnp.sum(t1 * sel_ref[...], axis=1, keepdims=True)
    inv = 1.0 / jnp.maximum(cnt, 1.0)
    o_ref[0] = p_ref[0] * inv
    o_ref[1] = t1 * inv * mask_ref[...] + oh_ref[...]


def _combine_mid(p, sel1, mask1, oh1):
    return pl.pallas_call(
        _c1_body,
        grid=(10,),
        in_specs=[
            pl.BlockSpec((2, 1000, HW), lambda i: (0, i, 0)),
            pl.BlockSpec((1, HW), lambda i: (0, 0)),
            pl.BlockSpec((1, HW), lambda i: (0, 0)),
            pl.BlockSpec((1, HW), lambda i: (0, 0)),
        ],
        out_specs=pl.BlockSpec((2, 1000, HW), lambda i: (0, i, 0)),
        out_shape=jax.ShapeDtypeStruct((2, N_NODES, HW), jnp.float32),
    )(p, sel1, mask1, oh1)


def _c2_body(q_ref, sel_ref, sm0_ref, sm1_ref, b_ref, o_ref):
    t0 = q_ref[0]
    t1 = q_ref[1]
    cnt = jnp.sum(t1 * sel_ref[...], axis=1, keepdims=True)
    inv = 1.0 / jnp.maximum(cnt, 1.0)
    o_ref[...] = (
        jnp.dot(t0 * inv, sm0_ref[...], preferred_element_type=jnp.float32,
                precision=lax.Precision.HIGHEST)
        + jnp.dot(t1 * inv, sm1_ref[...], preferred_element_type=jnp.float32,
                  precision=lax.Precision.HIGHEST)
        + b_ref[...]
    )


def _combine_out(q, sel1, sm0, sm1, b2):
    return pl.pallas_call(
        _c2_body,
        grid=(10,),
        in_specs=[
            pl.BlockSpec((2, 1000, HW), lambda i: (0, i, 0)),
            pl.BlockSpec((1, HW), lambda i: (0, 0)),
            pl.BlockSpec((HW, D), lambda i: (0, 0)),
            pl.BlockSpec((HW, D), lambda i: (0, 0)),
            pl.BlockSpec((1, D), lambda i: (0, 0)),
        ],
        out_specs=pl.BlockSpec((1000, D), lambda i: (i, 0)),
        out_shape=jax.ShapeDtypeStruct((N_NODES, D), jnp.float32),
    )(q, sel1, sm0, sm1, b2)


def kernel(x, hyperedge_index, temb, W, b):
    del temb
    hidx = hyperedge_index.astype(jnp.int32)
    nidx = hidx[0].reshape(NT, NBLK, BLK)
    eidx = hidx[1].reshape(NT, NBLK, BLK)

    w_ext = jnp.pad(W, ((0, 0), (0, DP - D)))
    oh = jnp.zeros((1, DP), jnp.float32).at[0, D].set(1.0)
    sel1 = jnp.zeros((1, HW), jnp.float32).at[0, CCOL].set(1.0)
    mask1 = jnp.pad(jnp.ones((1, CCOL), jnp.float32), ((0, 0), (0, HW - CCOL)))
    sm0 = jnp.eye(HW, D, dtype=jnp.float32)
    sm1 = jnp.eye(HW, D, k=HW, dtype=jnp.float32)
    zeros = jnp.zeros((CH, HW), jnp.float32)

    xw = _matmul_ext(x, w_ext, oh)
    xw3 = xw.reshape(N_NODES, 2, HW).swapaxes(0, 1)
    p = _sc_pass(xw3, nidx, eidx, zeros).reshape(2, N_NODES, HW)
    ef = _combine_mid(p, sel1, mask1, sel1)
    q = _sc_pass(ef, eidx, nidx, zeros).reshape(2, N_NODES, HW)
    return _combine_out(q, sel1, sm0, sm1, b.reshape(1, D))


# speedup vs baseline: 29.5699x; 1.4516x over previous
import jax
import jax.numpy as jnp
from jax import lax
from jax.experimental import pallas as pl
from jax.experimental.pallas import tpu as pltpu
from jax.experimental.pallas import tpu_sc as plsc

N_NODES = 10000
N_INC = 320000
D = 128
DP = 160
HW = 80
CCOL = 48

NT = 16
PER_T = N_INC // NT
BLK = 80
NBLK = PER_T // BLK
NBUF = 5
STRIPE = N_NODES // NT
CH = 25

_MESH = plsc.VectorSubcoreMesh(core_axis_name="c", subcore_axis_name="s")


def _mm_body(x_ref, w_ref, oh_ref, o_ref):
    o_ref[0] = (
        jnp.dot(x_ref[...], w_ref[0], preferred_element_type=jnp.float32)
        + oh_ref[0]
    )


def _matmul_ext(x, w_ext, oh):
    return pl.pallas_call(
        _mm_body,
        grid=(2, 10),
        in_specs=[
            pl.BlockSpec((1000, D), lambda j, i: (i, 0)),
            pl.BlockSpec((1, D, HW), lambda j, i: (j, 0, 0)),
            pl.BlockSpec((1, 1, HW), lambda j, i: (j, 0, 0)),
        ],
        out_specs=pl.BlockSpec((1, 1000, HW), lambda j, i: (j, i, 0)),
        out_shape=jax.ShapeDtypeStruct((2, N_NODES, HW), jnp.float32),
    )(x, w_ext, oh)


def _sc_body(table, gidx, sidx, zeros, out,
             gidx_v, sidx_v, r0, r1, r2, r3, r4, wb_v, acc,
             s0, s1, s2, s3, s4):
    rows = (r0, r1, r2, r3, r4)
    sems = (s0, s1, s2, s3, s4)
    c = lax.axis_index("c")
    s = lax.axis_index("s")
    half = table.at[c]

    pltpu.sync_copy(zeros, wb_v)
    for k in range(STRIPE // CH):
        pltpu.sync_copy(wb_v, acc.at[pl.ds(s * STRIPE + k * CH, CH)])

    pltpu.sync_copy(gidx.at[s], gidx_v)
    pltpu.sync_copy(sidx.at[s], sidx_v)
    plsc.subcore_barrier()

    for b in range(NBUF):
        pltpu.async_copy(half.at[gidx_v.at[b]], rows[b], sems[b])

    def grp(g, carry):
        base = g * NBUF
        for b in range(NBUF):
            j = base + b
            pltpu.make_async_copy(half.at[gidx_v.at[j]], rows[b], sems[b]).wait()
            pltpu.sync_copy(rows[b], acc.at[sidx_v.at[j]], add=True)
            pltpu.async_copy(half.at[gidx_v.at[j + NBUF]], rows[b], sems[b])
        return carry

    lax.fori_loop(0, NBLK // NBUF - 1, grp, 0)
    for b in range(NBUF):
        j = NBLK - NBUF + b
        pltpu.make_async_copy(half.at[gidx_v.at[j]], rows[b], sems[b]).wait()
        pltpu.sync_copy(rows[b], acc.at[sidx_v.at[j]], add=True)
    plsc.subcore_barrier()

    for k in range(STRIPE // CH):
        pltpu.sync_copy(acc.at[pl.ds(s * STRIPE + k * CH, CH)], wb_v)
        pltpu.sync_copy(wb_v, out.at[c, s, pl.ds(k * CH, CH)])


def _sc_pass(table, gidx, sidx, zeros):
    return pl.kernel(
        _sc_body,
        out_type=jax.ShapeDtypeStruct((2, NT, STRIPE, HW), jnp.float32),
        mesh=_MESH,
        scratch_types=[
            pltpu.VMEM((NBLK, BLK), jnp.int32),
            pltpu.VMEM((NBLK, BLK), jnp.int32),
        ] + [pltpu.VMEM((BLK, HW), jnp.float32) for _ in range(NBUF)] + [
            pltpu.VMEM((CH, HW), jnp.float32),
            pltpu.VMEM_SHARED((N_NODES, HW), jnp.float32),
        ] + [pltpu.SemaphoreType.DMA for _ in range(NBUF)],
        compiler_params=pltpu.CompilerParams(use_tc_tiling_on_sc=False),
    )(table, gidx, sidx, zeros)


def _c1_body(p_ref, sel_ref, mask_ref, oh_ref, o_ref):
    t1 = p_ref[1]
    cnt = jnp.sum(t1 * sel_ref[...], axis=1, keepdims=True)
    inv = 1.0 / jnp.maximum(cnt, 1.0)
    o_ref[0] = p_ref[0] * inv
    o_ref[1] = t1 * inv * mask_ref[...] + oh_ref[...]


def _combine_mid(p, sel1, mask1, oh1):
    return pl.pallas_call(
        _c1_body,
        grid=(10,),
        in_specs=[
            pl.BlockSpec((2, 1000, HW), lambda i: (0, i, 0)),
            pl.BlockSpec((1, HW), lambda i: (0, 0)),
            pl.BlockSpec((1, HW), lambda i: (0, 0)),
            pl.BlockSpec((1, HW), lambda i: (0, 0)),
        ],
        out_specs=pl.BlockSpec((2, 1000, HW), lambda i: (0, i, 0)),
        out_shape=jax.ShapeDtypeStruct((2, N_NODES, HW), jnp.float32),
    )(p, sel1, mask1, oh1)


def _c2_body(q_ref, sel_ref, sm0_ref, sm1_ref, b_ref, o_ref):
    t0 = q_ref[0]
    t1 = q_ref[1]
    cnt = jnp.sum(t1 * sel_ref[...], axis=1, keepdims=True)
    inv = 1.0 / jnp.maximum(cnt, 1.0)
    o_ref[...] = (
        jnp.dot(t0 * inv, sm0_ref[...], preferred_element_type=jnp.float32,
                precision=lax.Precision.HIGHEST)
        + jnp.dot(t1 * inv, sm1_ref[...], preferred_element_type=jnp.float32,
                  precision=lax.Precision.HIGHEST)
        + b_ref[...]
    )


def _combine_out(q, sel1, sm0, sm1, b2):
    return pl.pallas_call(
        _c2_body,
        grid=(10,),
        in_specs=[
            pl.BlockSpec((2, 1000, HW), lambda i: (0, i, 0)),
            pl.BlockSpec((1, HW), lambda i: (0, 0)),
            pl.BlockSpec((HW, D), lambda i: (0, 0)),
            pl.BlockSpec((HW, D), lambda i: (0, 0)),
            pl.BlockSpec((1, D), lambda i: (0, 0)),
        ],
        out_specs=pl.BlockSpec((1000, D), lambda i: (i, 0)),
        out_shape=jax.ShapeDtypeStruct((N_NODES, D), jnp.float32),
    )(q, sel1, sm0, sm1, b2)


def kernel(x, hyperedge_index, temb, W, b):
    del temb
    hidx = hyperedge_index.astype(jnp.int32)
    nidx = hidx[0].reshape(NT, NBLK, BLK)
    eidx = hidx[1].reshape(NT, NBLK, BLK)

    w_ext = jnp.pad(W, ((0, 0), (0, DP - D)))
    w3 = w_ext.reshape(D, 2, HW).swapaxes(0, 1)
    oh = jnp.zeros((2, 1, HW), jnp.float32).at[1, 0, CCOL].set(1.0)
    sel1 = jnp.zeros((1, HW), jnp.float32).at[0, CCOL].set(1.0)
    mask1 = jnp.pad(jnp.ones((1, CCOL), jnp.float32), ((0, 0), (0, HW - CCOL)))
    sm0 = jnp.eye(HW, D, dtype=jnp.float32)
    sm1 = jnp.eye(HW, D, k=HW, dtype=jnp.float32)
    zeros = jnp.zeros((CH, HW), jnp.float32)

    xw3 = _matmul_ext(x, w3, oh)
    p = _sc_pass(xw3, nidx, eidx, zeros).reshape(2, N_NODES, HW)
    ef = _combine_mid(p, sel1, mask1, sel1)
    q = _sc_pass(ef, eidx, nidx, zeros).reshape(2, N_NODES, HW)
    return _combine_out(q, sel1, sm0, sm1, b.reshape(1, D))


# speedup vs baseline: 33.8744x; 1.1456x over previous
import jax
import jax.numpy as jnp
from jax import lax
from jax.experimental import pallas as pl
from jax.experimental.pallas import tpu as pltpu
from jax.experimental.pallas import tpu_sc as plsc

N_NODES = 10000
N_INC = 320000
D = 128
HF = 64
HW = 80
CCOL = 64

NT = 16
PER_T = N_INC // NT
BLK = 80
NBLK = PER_T // BLK
NBUF = 5
STRIPE = N_NODES // NT
CH = 25

_MESH = plsc.VectorSubcoreMesh(core_axis_name="c", subcore_axis_name="s")


def _mm_body(x_ref, w_ref, oh_ref, o_ref):
    o_ref[0] = (
        jnp.dot(x_ref[...], w_ref[0], preferred_element_type=jnp.float32)
        + oh_ref[0]
    )


def _matmul_ext(x, w3, oh):
    return pl.pallas_call(
        _mm_body,
        grid=(2, 10),
        in_specs=[
            pl.BlockSpec((1000, D), lambda j, i: (i, 0)),
            pl.BlockSpec((1, D, HW), lambda j, i: (j, 0, 0)),
            pl.BlockSpec((1, 1, HW), lambda j, i: (j, 0, 0)),
        ],
        out_specs=pl.BlockSpec((1, 1000, HW), lambda j, i: (j, i, 0)),
        out_shape=jax.ShapeDtypeStruct((2, N_NODES, HW), jnp.float32),
    )(x, w3, oh)


def _make_sc_body(final):
    def _sc_body(table, gidx, sidx, zeros, bvec, out,
                 gidx_v, sidx_v, r0, r1, r2, r3, r4, wb_v, wf_v, bias_v, acc,
                 s0, s1, s2, s3, s4):
        rows = (r0, r1, r2, r3, r4)
        sems = (s0, s1, s2, s3, s4)
        c = lax.axis_index("c")
        s = lax.axis_index("s")
        half = table.at[c]

        pltpu.sync_copy(zeros, wb_v)

        def zero_chunk(k, carry):
            pltpu.sync_copy(wb_v, acc.at[pl.ds(s * STRIPE + k * CH, CH)])
            return carry

        lax.fori_loop(0, STRIPE // CH, zero_chunk, 0)

        pltpu.sync_copy(gidx.at[s], gidx_v)
        pltpu.sync_copy(sidx.at[s], sidx_v)
        pltpu.sync_copy(bvec.at[c], bias_v)
        plsc.subcore_barrier()

        for b in range(NBUF):
            pltpu.async_copy(half.at[gidx_v.at[b]], rows[b], sems[b])

        def grp(g, carry):
            base = g * NBUF
            for b in range(NBUF):
                j = base + b
                pltpu.make_async_copy(
                    half.at[gidx_v.at[j]], rows[b], sems[b]).wait()
                pltpu.sync_copy(rows[b], acc.at[sidx_v.at[j]], add=True)
                pltpu.async_copy(
                    half.at[gidx_v.at[j + NBUF]], rows[b], sems[b])
            return carry

        lax.fori_loop(0, NBLK // NBUF - 1, grp, 0)
        for b in range(NBUF):
            j = NBLK - NBUF + b
            pltpu.make_async_copy(half.at[gidx_v.at[j]], rows[b], sems[b]).wait()
            pltpu.sync_copy(rows[b], acc.at[sidx_v.at[j]], add=True)
        plsc.subcore_barrier()

        ccol16 = jnp.full((16,), CCOL, jnp.int32)

        def norm_chunk(k, carry):
            base = s * STRIPE + k * CH
            pltpu.sync_copy(acc.at[pl.ds(base, CH)], wb_v)

            for r in range(CH):
                bc = plsc.load_gather(wb_v, [jnp.full((16,), r, jnp.int32), ccol16])
                inv = 1.0 / jnp.maximum(bc, 1.0)
                if final:
                    for q in range(HF // 16):
                        wf_v[r, pl.ds(q * 16, 16)] = (
                            wb_v[r, pl.ds(q * 16, 16)] * inv
                            + bias_v[pl.ds(q * 16, 16)]
                        )
                else:
                    for q in range(HW // 16):
                        wb_v[r, pl.ds(q * 16, 16)] = (
                            wb_v[r, pl.ds(q * 16, 16)] * inv
                        )
            if final:
                pltpu.sync_copy(
                    wf_v, out.at[pl.ds(base, CH), pl.ds(c * HF, HF)])
            else:
                pltpu.sync_copy(wb_v, out.at[c, s, pl.ds(k * CH, CH)])
            return carry

        lax.fori_loop(0, STRIPE // CH, norm_chunk, 0)

    return _sc_body


_OUT_MID = jax.ShapeDtypeStruct((2, NT, STRIPE, HW), jnp.float32)
_OUT_FINAL = jax.ShapeDtypeStruct((N_NODES, D), jnp.float32)


def _sc_pass(table, gidx, sidx, zeros, bvec, final):
    return pl.kernel(
        _make_sc_body(final),
        out_type=_OUT_FINAL if final else _OUT_MID,
        mesh=_MESH,
        scratch_types=[
            pltpu.VMEM((NBLK, BLK), jnp.int32),
            pltpu.VMEM((NBLK, BLK), jnp.int32),
        ] + [pltpu.VMEM((BLK, HW), jnp.float32) for _ in range(NBUF)] + [
            pltpu.VMEM((CH, HW), jnp.float32),
            pltpu.VMEM((CH, HF), jnp.float32),
            pltpu.VMEM((HW,), jnp.float32),
            pltpu.VMEM_SHARED((N_NODES, HW), jnp.float32),
        ] + [pltpu.SemaphoreType.DMA for _ in range(NBUF)],
        compiler_params=pltpu.CompilerParams(
            use_tc_tiling_on_sc=False, needs_layout_passes=False),
    )(table, gidx, sidx, zeros, bvec)


def kernel(x, hyperedge_index, temb, W, b):
    del temb
    hidx = hyperedge_index.astype(jnp.int32)
    nidx = hidx[0].reshape(NT, NBLK, BLK)
    eidx = hidx[1].reshape(NT, NBLK, BLK)

    w3 = jnp.pad(W.reshape(D, 2, HF).swapaxes(0, 1), ((0, 0), (0, 0), (0, HW - HF)))
    oh = jnp.zeros((2, 1, HW), jnp.float32).at[:, 0, CCOL].set(1.0)
    bvec = jnp.pad(b.reshape(2, HF), ((0, 0), (0, HW - HF)))
    zvec = jnp.zeros((2, HW), jnp.float32)
    zeros = jnp.zeros((CH, HW), jnp.float32)

    xw3 = _matmul_ext(x, w3, oh)
    ef = _sc_pass(xw3, nidx, eidx, zeros, zvec, final=False)
    ef3 = ef.reshape(2, N_NODES, HW)
    return _sc_pass(ef3, eidx, nidx, zeros, bvec, final=True)


# speedup vs baseline: 36.5239x; 1.0782x over previous
import jax
import jax.numpy as jnp
from jax import lax
from jax.experimental import pallas as pl
from jax.experimental.pallas import tpu as pltpu
from jax.experimental.pallas import tpu_sc as plsc

N_NODES = 10000
N_INC = 320000
D = 128
HF = 64

NT = 16
PER_T = N_INC // NT
BLK = 80
NBLK = PER_T // BLK
NBUF = 5
NBINS = 10240
STRIPE = NBINS // NT
CH = 20

_MESH = plsc.VectorSubcoreMesh(core_axis_name="c", subcore_axis_name="s")


def _mm_body(x_ref, w_ref, o_ref):
    o_ref[0] = jnp.dot(x_ref[...], w_ref[0], preferred_element_type=jnp.float32)


def _matmul_ext(x, w3):
    return pl.pallas_call(
        _mm_body,
        grid=(2, 10),
        in_specs=[
            pl.BlockSpec((1000, D), lambda j, i: (i, 0)),
            pl.BlockSpec((1, D, HF), lambda j, i: (j, 0, 0)),
        ],
        out_specs=pl.BlockSpec((1, 1000, HF), lambda j, i: (j, i, 0)),
        out_shape=jax.ShapeDtypeStruct((2, N_NODES, HF), jnp.float32),
    )(x, w3)


def _make_sc_body(final):
    def _sc_body(table, gidx, sidx, zeros, bvec, out,
                 gidx_v, sidx_v, r0, r1, r2, r3, r4, wb_v, bias_v,
                 hist_v, cnt_v, tmp_v, acc, hsh,
                 s0, s1, s2, s3, s4):
        rows = (r0, r1, r2, r3, r4)
        sems = (s0, s1, s2, s3, s4)
        c = lax.axis_index("c")
        s = lax.axis_index("s")
        half = table.at[c]
        ones16 = jnp.full((16,), 1.0, jnp.float32)
        zero16 = jnp.zeros((16,), jnp.float32)

        pltpu.sync_copy(zeros, wb_v)

        def zero_chunk(k, carry):
            pltpu.sync_copy(wb_v, acc.at[pl.ds(s * STRIPE + k * CH, CH)])
            return carry

        lax.fori_loop(0, STRIPE // CH, zero_chunk, 0)

        def zero_hist(k, carry):
            hist_v[pl.ds(k * 16, 16)] = zero16
            return carry

        lax.fori_loop(0, NBINS // 16, zero_hist, 0)

        pltpu.sync_copy(gidx.at[s], gidx_v)
        pltpu.sync_copy(sidx.at[s], sidx_v)
        pltpu.sync_copy(bvec.at[c], bias_v)
        plsc.subcore_barrier()

        for b in range(NBUF):
            pltpu.async_copy(half.at[gidx_v.at[b]], rows[b], sems[b])

        def hist_block(j):
            for q in range(BLK // 16):
                idx16 = sidx_v[j, pl.ds(q * 16, 16)]
                plsc.addupdate_scatter(hist_v, [idx16], ones16)

        def grp(g, carry):
            base = g * NBUF
            for b in range(NBUF):
                j = base + b
                pltpu.make_async_copy(
                    half.at[gidx_v.at[j]], rows[b], sems[b]).wait()
                pltpu.sync_copy(rows[b], acc.at[sidx_v.at[j]], add=True)
                pltpu.async_copy(
                    half.at[gidx_v.at[j + NBUF]], rows[b], sems[b])
                hist_block(j)
            return carry

        lax.fori_loop(0, NBLK // NBUF - 1, grp, 0)
        for b in range(NBUF):
            j = NBLK - NBUF + b
            pltpu.make_async_copy(half.at[gidx_v.at[j]], rows[b], sems[b]).wait()
            pltpu.sync_copy(rows[b], acc.at[sidx_v.at[j]], add=True)
            hist_block(j)

        pltpu.sync_copy(hist_v, hsh.at[s])
        plsc.subcore_barrier()
        pltpu.sync_copy(hsh.at[0, pl.ds(s * STRIPE, STRIPE)], cnt_v)
        for t in range(1, NT):
            pltpu.sync_copy(hsh.at[t, pl.ds(s * STRIPE, STRIPE)], tmp_v)

            def madd(k, carry):
                cnt_v[pl.ds(k * 16, 16)] = (
                    cnt_v[pl.ds(k * 16, 16)] + tmp_v[pl.ds(k * 16, 16)]
                )
                return carry

            lax.fori_loop(0, STRIPE // 16, madd, 0)

        def norm_chunk(k, carry):
            base = s * STRIPE + k * CH
            pltpu.sync_copy(acc.at[pl.ds(base, CH)], wb_v)
            for r in range(CH):
                bc = plsc.load_gather(cnt_v, [k * CH + r + jnp.zeros((16,), jnp.int32)])
                inv = 1.0 / jnp.maximum(bc, 1.0)
                for q in range(HF // 16):
                    if final:
                        wb_v[r, pl.ds(q * 16, 16)] = (
                            wb_v[r, pl.ds(q * 16, 16)] * inv
                            + bias_v[pl.ds(q * 16, 16)]
                        )
                    else:
                        wb_v[r, pl.ds(q * 16, 16)] = (
                            wb_v[r, pl.ds(q * 16, 16)] * inv
                        )
            if final:
                pltpu.sync_copy(
                    wb_v, out.at[pl.ds(base, CH), pl.ds(c * HF, HF)])
            else:
                pltpu.sync_copy(wb_v, out.at[c, s, pl.ds(k * CH, CH)])
            return carry

        if final:
            lax.fori_loop(
                0,
                lax.select(s == NT - 1, (N_NODES - (NT - 1) * STRIPE) // CH,
                           STRIPE // CH),
                norm_chunk, 0)
        else:
            lax.fori_loop(0, STRIPE // CH, norm_chunk, 0)

    return _sc_body


_OUT_MID = jax.ShapeDtypeStruct((2, NT, STRIPE, HF), jnp.float32)
_OUT_FINAL = jax.ShapeDtypeStruct((N_NODES, D), jnp.float32)


def _sc_pass(table, gidx, sidx, zeros, bvec, final):
    return pl.kernel(
        _make_sc_body(final),
        out_type=_OUT_FINAL if final else _OUT_MID,
        mesh=_MESH,
        scratch_types=[
            pltpu.VMEM((NBLK, BLK), jnp.int32),
            pltpu.VMEM((NBLK, BLK), jnp.int32),
        ] + [pltpu.VMEM((BLK, HF), jnp.float32) for _ in range(NBUF)] + [
            pltpu.VMEM((CH, HF), jnp.float32),
            pltpu.VMEM((HF,), jnp.float32),
            pltpu.VMEM((NBINS,), jnp.float32),
            pltpu.VMEM((STRIPE,), jnp.float32),
            pltpu.VMEM((STRIPE,), jnp.float32),
            pltpu.VMEM_SHARED((NBINS, HF), jnp.float32),
            pltpu.VMEM_SHARED((NT, NBINS), jnp.float32),
        ] + [pltpu.SemaphoreType.DMA for _ in range(NBUF)],
        compiler_params=pltpu.CompilerParams(
            use_tc_tiling_on_sc=False, needs_layout_passes=False),
    )(table, gidx, sidx, zeros, bvec)


def kernel(x, hyperedge_index, temb, W, b):
    del temb
    hidx = hyperedge_index.astype(jnp.int32)
    nidx = hidx[0].reshape(NT, NBLK, BLK)
    eidx = hidx[1].reshape(NT, NBLK, BLK)

    w3 = W.reshape(D, 2, HF).swapaxes(0, 1)
    bvec = b.reshape(2, HF)
    zeros = jnp.zeros((CH, HF), jnp.float32)

    xw3 = _matmul_ext(x, w3)
    ef = _sc_pass(xw3, nidx, eidx, zeros, bvec, final=False)
    ef3 = ef.reshape(2, NBINS, HF)
    return _sc_pass(ef3, eidx, nidx, zeros, bvec, final=True)
